# baseline (speedup 1.0000x reference)
import functools
import jax
import jax.numpy as jnp
from jax import lax
from jax.experimental import pallas as pl
from jax.experimental.pallas import tpu as pltpu
from jax.experimental.pallas import tpu_sc as plsc

PAD_ID = 0
EOS_ID = 2
BATCH = 4096
SEQ = 20
DIM = 64

NUM_CORES = 2
NUM_SUBCORES = 16
NW = NUM_CORES * NUM_SUBCORES
TOTAL = BATCH * SEQ
ROWS_PER_W = TOTAL // NW
LANES = 16
CHUNK = 256
VPC = CHUNK // LANES
NCHUNK = ROWS_PER_W // CHUNK
NBUF = 2
NITER = NCHUNK // NBUF


_mesh = plsc.VectorSubcoreMesh(
    core_axis_name="c", subcore_axis_name="s",
    num_cores=NUM_CORES, num_subcores=NUM_SUBCORES)


@functools.partial(
    pl.kernel,
    mesh=_mesh,
    out_type=jax.ShapeDtypeStruct((TOTAL, DIM), jnp.float32),
    scratch_types=[
        pltpu.VMEM((ROWS_PER_W,), jnp.int32),
        pltpu.VMEM((NBUF, CHUNK, DIM), jnp.float32),
        pltpu.SemaphoreType.DMA((NBUF,)),
        pltpu.SemaphoreType.DMA((NBUF,)),
    ],
    compiler_params=pltpu.CompilerParams(use_tc_tiling_on_sc=False),
)
def _sc_gather(ids_hbm, table_hbm, out_hbm, idx_v, rows_v, gsem, osem):
    wid = lax.axis_index("s") * NUM_CORES + lax.axis_index("c")
    base = wid * ROWS_PER_W
    pltpu.sync_copy(ids_hbm.at[pl.ds(base, ROWS_PER_W)], idx_v)

    def start_gathers(c, b):
        for k in range(VPC):
            vec = idx_v[pl.ds(c * CHUNK + k * LANES, LANES)]
            pltpu.async_copy(
                table_hbm.at[vec],
                rows_v.at[b, pl.ds(k * LANES, LANES)],
                gsem.at[b],
            )

    def drain_gathers(b):
        for k in range(VPC):
            pltpu.make_async_copy(
                table_hbm.at[idx_v[pl.ds(0, LANES)]],
                rows_v.at[b, pl.ds(0, LANES)],
                gsem.at[b],
            ).wait()

    def start_out(c, b):
        pltpu.async_copy(
            rows_v.at[b], out_hbm.at[pl.ds(base + c * CHUNK, CHUNK)],
            osem.at[b])

    def out_done(b):
        pltpu.make_async_copy(
            rows_v.at[b], out_hbm.at[pl.ds(base, CHUNK)], osem.at[b]).wait()

    def loop_body(t, carry):
        c0 = t * NBUF

        @pl.when(t > 0)
        def _():
            for b in range(NBUF):
                out_done(b)

        for b in range(NBUF):
            start_gathers(c0 + b, b)
        for b in range(NBUF):
            drain_gathers(b)
            start_out(c0 + b, b)
        return carry

    lax.fori_loop(0, NITER, loop_body, 0)
    for b in range(NBUF):
        out_done(b)


GB = 256


def _finish_body(rows_hbm, ids_ref, mat_ref, mask_ref, eos_ref, stage, sem):
    j = pl.program_id(0)
    for s in range(SEQ):
        pltpu.async_copy(
            rows_hbm.at[pl.ds(s * BATCH + j * GB, GB)], stage.at[:, s], sem)
    for s in range(SEQ):
        pltpu.make_async_copy(
            rows_hbm.at[pl.ds(s * BATCH + j * GB, GB)],
            stage.at[:, s], sem).wait()
    ids = ids_ref[...]
    mat_ref[...] = stage[...]
    mask_ref[...] = ids == PAD_ID
    eos_ref[...] = (ids == EOS_ID).astype(jnp.float32)


_finish_call = pl.pallas_call(
    _finish_body,
    grid=(BATCH // GB,),
    in_specs=[
        pl.BlockSpec(memory_space=pltpu.MemorySpace.HBM),
        pl.BlockSpec((GB, SEQ), lambda j: (j, 0)),
    ],
    out_specs=(
        pl.BlockSpec((GB, SEQ, DIM), lambda j: (j, 0, 0)),
        pl.BlockSpec((GB, SEQ), lambda j: (j, 0)),
        pl.BlockSpec((GB, SEQ), lambda j: (j, 0)),
    ),
    out_shape=(
        jax.ShapeDtypeStruct((BATCH, SEQ, DIM), jnp.float32),
        jax.ShapeDtypeStruct((BATCH, SEQ), jnp.bool_),
        jax.ShapeDtypeStruct((BATCH, SEQ), jnp.float32),
    ),
    scratch_shapes=[
        pltpu.VMEM((GB, SEQ, DIM), jnp.float32),
        pltpu.SemaphoreType.DMA,
    ],
)


def kernel(lookup_ids, embedding_matrix):
    flat_t = jnp.transpose(lookup_ids).reshape(-1)
    gathered = _sc_gather(flat_t, embedding_matrix)
    matrices, mask, eos = _finish_call(gathered, lookup_ids)
    return (matrices, mask, eos)


# speedup vs baseline: 1.1005x; 1.1005x over previous
import functools
import jax
import jax.numpy as jnp
from jax import lax
from jax.experimental import pallas as pl
from jax.experimental.pallas import tpu as pltpu
from jax.experimental.pallas import tpu_sc as plsc

PAD_ID = 0
EOS_ID = 2
BATCH = 4096
SEQ = 20
DIM = 64

NUM_CORES = 2
NUM_SUBCORES = 16
NW = NUM_CORES * NUM_SUBCORES
TOTAL = BATCH * SEQ
ROWS_PER_W = TOTAL // NW
CHUNK = 640
NCHUNK = ROWS_PER_W // CHUNK
NBUF = 2


_mesh = plsc.VectorSubcoreMesh(
    core_axis_name="c", subcore_axis_name="s",
    num_cores=NUM_CORES, num_subcores=NUM_SUBCORES)


@functools.partial(
    pl.kernel,
    mesh=_mesh,
    out_type=jax.ShapeDtypeStruct((TOTAL, DIM), jnp.float32),
    scratch_types=[
        pltpu.VMEM((NCHUNK, CHUNK), jnp.int32),
        pltpu.VMEM((NBUF, CHUNK, DIM), jnp.float32),
        pltpu.SemaphoreType.DMA((NBUF,)),
        pltpu.SemaphoreType.DMA((NBUF,)),
    ],
    compiler_params=pltpu.CompilerParams(use_tc_tiling_on_sc=False),
)
def _sc_gather(ids_hbm, table_hbm, out_hbm, idx_v, rows_v, gsem, osem):
    wid = lax.axis_index("s") * NUM_CORES + lax.axis_index("c")
    base = wid * ROWS_PER_W
    pltpu.sync_copy(ids_hbm.at[wid], idx_v)

    def start_gather(j):
        pltpu.async_copy(
            table_hbm.at[idx_v.at[j]], rows_v.at[j % NBUF], gsem.at[j % NBUF])

    def gather_done(j):
        pltpu.make_async_copy(
            table_hbm.at[idx_v.at[j]], rows_v.at[j % NBUF],
            gsem.at[j % NBUF]).wait()

    def start_out(j):
        pltpu.async_copy(
            rows_v.at[j % NBUF],
            out_hbm.at[pl.ds(base + j * CHUNK, CHUNK)], osem.at[j % NBUF])

    def out_done(j):
        pltpu.make_async_copy(
            rows_v.at[j % NBUF],
            out_hbm.at[pl.ds(base + j * CHUNK, CHUNK)],
            osem.at[j % NBUF]).wait()

    for j in range(min(NBUF, NCHUNK)):
        start_gather(j)
    for j in range(NCHUNK):
        gather_done(j)
        start_out(j)
        nxt = j + NBUF
        if nxt < NCHUNK:
            out_done(j)
            start_gather(nxt)
    for j in range(max(NCHUNK - NBUF, 0), NCHUNK):
        out_done(j)


def _mask_body(ids_ref, mask_ref, eos_ref):
    ids = ids_ref[...]
    mask_ref[...] = ids == PAD_ID
    eos_ref[...] = (ids == EOS_ID).astype(jnp.float32)


_mask_call = pl.pallas_call(
    _mask_body,
    out_shape=(
        jax.ShapeDtypeStruct((TOTAL // 128, 128), jnp.bool_),
        jax.ShapeDtypeStruct((TOTAL // 128, 128), jnp.float32),
    ),
)


def kernel(lookup_ids, embedding_matrix):
    flat = lookup_ids.reshape(-1)
    ids_sc = flat.reshape(NW, NCHUNK, CHUNK)
    gathered = _sc_gather(ids_sc, embedding_matrix)
    matrices = gathered.reshape(BATCH, SEQ, DIM)
    mask2d, eos2d = _mask_call(flat.reshape(TOTAL // 128, 128))
    mask = mask2d.reshape(BATCH, SEQ)
    eos = eos2d.reshape(BATCH, SEQ)
    return (matrices, mask, eos)
